# Initial kernel scaffold; baseline (speedup 1.0000x reference)
import jax
import jax.numpy as jnp
from jax.experimental import pallas as pl


def kernel(inputs, embedding, position_embedding):
    raise NotImplementedError("write your pallas kernel here")


# speedup vs baseline: 4.4261x; 4.4261x over previous
import functools

import jax
import jax.numpy as jnp
from jax import lax
from jax.experimental import pallas as pl
from jax.experimental.pallas import tpu as pltpu
from jax.experimental.pallas import tpu_sc as plsc

_VOCAB = 1000000
_HIDDEN = 64
_BATCH = 4096
_SEQ = 200

_info = plsc.get_sparse_core_info()
_NC, _NS = _info.num_cores, _info.num_subcores
_NW = _NC * _NS
_BPW = _BATCH // _NW
_HSEQ = _SEQ // 2


def _sc_body(idx_hbm, emb_hbm, pos_hbm, out_hbm, idx_v, pos_v, rows_v, sem):
    wid = lax.axis_index("s") * _NC + lax.axis_index("c")
    pltpu.sync_copy(idx_hbm.at[wid], idx_v)
    pltpu.sync_copy(pos_hbm.at[pl.ds(0, _SEQ)], pos_v)
    base0 = wid * (_BPW * _SEQ)

    def body(j, carry):
        g0 = pltpu.async_copy(emb_hbm.at[idx_v.at[j, 0]],
                              rows_v.at[pl.ds(0, _HSEQ)], sem)
        g1 = pltpu.async_copy(emb_hbm.at[idx_v.at[j, 1]],
                              rows_v.at[pl.ds(_HSEQ, _HSEQ)], sem)
        base = base0 + j * _SEQ
        pltpu.sync_copy(pos_v, out_hbm.at[pl.ds(base, _SEQ), pl.ds(0, _HIDDEN)])
        g0.wait()
        g1.wait()
        pltpu.sync_copy(rows_v,
                        out_hbm.at[pl.ds(base, _SEQ), pl.ds(_HIDDEN, _HIDDEN)])
        return carry

    lax.fori_loop(0, _BPW, body, 0)


@functools.partial(jax.jit, static_argnums=())
def _run(idx, embedding, position_embedding):
    mesh = plsc.VectorSubcoreMesh(core_axis_name="c", subcore_axis_name="s")
    kern = pl.kernel(
        _sc_body,
        mesh=mesh,
        compiler_params=pltpu.CompilerParams(use_tc_tiling_on_sc=False),
        out_type=jax.ShapeDtypeStruct((_BATCH * _SEQ, 2 * _HIDDEN), jnp.float32),
        scratch_types=[
            pltpu.VMEM((_BPW, 2, _HSEQ), jnp.int32),
            pltpu.VMEM((_SEQ, _HIDDEN), jnp.float32),
            pltpu.VMEM((_SEQ, _HIDDEN), jnp.float32),
            pltpu.SemaphoreType.DMA,
        ],
    )
    return kern(idx, embedding, position_embedding)


def kernel(inputs, embedding, position_embedding):
    batch, seq = inputs.shape
    hidden = embedding.shape[1]
    idx = inputs.reshape(_NW, _BPW, 2, _HSEQ)
    out = _run(idx, embedding, position_embedding)
    return out.reshape(batch, seq, 2 * hidden)


# speedup vs baseline: 4.5273x; 1.0229x over previous
import functools

import jax
import jax.numpy as jnp
from jax import lax
from jax.experimental import pallas as pl
from jax.experimental.pallas import tpu as pltpu
from jax.experimental.pallas import tpu_sc as plsc

_VOCAB = 1000000
_HIDDEN = 64
_BATCH = 4096
_SEQ = 200

_info = plsc.get_sparse_core_info()
_NC, _NS = _info.num_cores, _info.num_subcores
_NW = _NC * _NS
_BPW = _BATCH // _NW
_HSEQ = _SEQ // 2


def _sc_body(idx_hbm, emb_hbm, pos_hbm, out_hbm,
             idx_v, pos_v, rows, gsem, wsem, psem):
    wid = lax.axis_index("s") * _NC + lax.axis_index("c")
    pltpu.sync_copy(idx_hbm.at[wid], idx_v)
    pltpu.sync_copy(pos_hbm.at[pl.ds(0, _SEQ)], pos_v)
    base0 = wid * (_BPW * _SEQ)

    def body(j, carry):
        slot = j % 2
        base = base0 + j * _SEQ

        @pl.when(j >= 2)
        def _():
            pltpu.make_async_copy(
                rows.at[slot],
                out_hbm.at[pl.ds(base - 2 * _SEQ, _SEQ),
                           pl.ds(_HIDDEN, _HIDDEN)],
                wsem.at[slot]).wait()
            pltpu.make_async_copy(
                pos_v,
                out_hbm.at[pl.ds(base - 2 * _SEQ, _SEQ), pl.ds(0, _HIDDEN)],
                psem.at[slot]).wait()

        g0 = pltpu.async_copy(emb_hbm.at[idx_v.at[j, 0]],
                              rows.at[slot, pl.ds(0, _HSEQ)], gsem)
        g1 = pltpu.async_copy(emb_hbm.at[idx_v.at[j, 1]],
                              rows.at[slot, pl.ds(_HSEQ, _HSEQ)], gsem)
        pltpu.make_async_copy(
            pos_v, out_hbm.at[pl.ds(base, _SEQ), pl.ds(0, _HIDDEN)],
            psem.at[slot]).start()
        g0.wait()
        g1.wait()
        pltpu.make_async_copy(
            rows.at[slot],
            out_hbm.at[pl.ds(base, _SEQ), pl.ds(_HIDDEN, _HIDDEN)],
            wsem.at[slot]).start()
        return carry

    lax.fori_loop(0, _BPW, body, 0)

    for j in (_BPW - 2, _BPW - 1):
        base = base0 + j * _SEQ
        pltpu.make_async_copy(
            rows.at[j % 2],
            out_hbm.at[pl.ds(base, _SEQ), pl.ds(_HIDDEN, _HIDDEN)],
            wsem.at[j % 2]).wait()
        pltpu.make_async_copy(
            pos_v, out_hbm.at[pl.ds(base, _SEQ), pl.ds(0, _HIDDEN)],
            psem.at[j % 2]).wait()


@functools.partial(jax.jit, static_argnums=())
def _run(idx, embedding, position_embedding):
    mesh = plsc.VectorSubcoreMesh(core_axis_name="c", subcore_axis_name="s")
    kern = pl.kernel(
        _sc_body,
        mesh=mesh,
        compiler_params=pltpu.CompilerParams(use_tc_tiling_on_sc=False),
        out_type=jax.ShapeDtypeStruct((_BATCH * _SEQ, 2 * _HIDDEN), jnp.float32),
        scratch_types=[
            pltpu.VMEM((_BPW, 2, _HSEQ), jnp.int32),
            pltpu.VMEM((_SEQ, _HIDDEN), jnp.float32),
            pltpu.VMEM((2, _SEQ, _HIDDEN), jnp.float32),
            pltpu.SemaphoreType.DMA,
            pltpu.SemaphoreType.DMA((2,)),
            pltpu.SemaphoreType.DMA((2,)),
        ],
    )
    return kern(idx, embedding, position_embedding)


def kernel(inputs, embedding, position_embedding):
    batch, seq = inputs.shape
    hidden = embedding.shape[1]
    idx = inputs.reshape(_NW, _BPW, 2, _HSEQ)
    out = _run(idx, embedding, position_embedding)
    return out.reshape(batch, seq, 2 * hidden)


# speedup vs baseline: 4.5491x; 1.0048x over previous
import functools

import jax
import jax.numpy as jnp
from jax import lax
from jax.experimental import pallas as pl
from jax.experimental.pallas import tpu as pltpu
from jax.experimental.pallas import tpu_sc as plsc

_VOCAB = 1000000
_HIDDEN = 64
_BATCH = 4096
_SEQ = 200

_info = plsc.get_sparse_core_info()
_NC, _NS = _info.num_cores, _info.num_subcores
_NW = _NC * _NS
_BPW = _BATCH // _NW
_HSEQ = _SEQ // 2
_NSLOT = 4
_LOOKAHEAD = 2


def _sc_body(idx_hbm, emb_hbm, pos_hbm, out_hbm,
             idx_v, pos_v, rows, gsem, wsem, psem):
    wid = lax.axis_index("s") * _NC + lax.axis_index("c")
    pltpu.sync_copy(idx_hbm.at[wid], idx_v)
    pltpu.sync_copy(pos_hbm.at[pl.ds(0, _SEQ)], pos_v)
    base0 = wid * (_BPW * _SEQ)

    def gathers(j, slot):
        pltpu.make_async_copy(emb_hbm.at[idx_v.at[j, 0]],
                              rows.at[slot, pl.ds(0, _HSEQ)],
                              gsem.at[slot]).start()
        pltpu.make_async_copy(emb_hbm.at[idx_v.at[j, 1]],
                              rows.at[slot, pl.ds(_HSEQ, _HSEQ)],
                              gsem.at[slot]).start()

    def wait_gathers(j, slot):
        pltpu.make_async_copy(emb_hbm.at[idx_v.at[j, 0]],
                              rows.at[slot, pl.ds(0, _HSEQ)],
                              gsem.at[slot]).wait()
        pltpu.make_async_copy(emb_hbm.at[idx_v.at[j, 1]],
                              rows.at[slot, pl.ds(_HSEQ, _HSEQ)],
                              gsem.at[slot]).wait()

    def writes_start(j, slot):
        base = base0 + j * _SEQ
        pltpu.make_async_copy(
            pos_v, out_hbm.at[pl.ds(base, _SEQ), pl.ds(0, _HIDDEN)],
            psem.at[slot]).start()
        pltpu.make_async_copy(
            rows.at[slot],
            out_hbm.at[pl.ds(base, _SEQ), pl.ds(_HIDDEN, _HIDDEN)],
            wsem.at[slot]).start()

    def writes_wait(j, slot):
        base = base0 + j * _SEQ
        pltpu.make_async_copy(
            pos_v, out_hbm.at[pl.ds(base, _SEQ), pl.ds(0, _HIDDEN)],
            psem.at[slot]).wait()
        pltpu.make_async_copy(
            rows.at[slot],
            out_hbm.at[pl.ds(base, _SEQ), pl.ds(_HIDDEN, _HIDDEN)],
            wsem.at[slot]).wait()

    for j in range(_LOOKAHEAD):
        gathers(j, j % _NSLOT)

    def body(j, carry):
        slot = j % _NSLOT
        wait_gathers(j, slot)
        writes_start(j, slot)

        @pl.when(j + _LOOKAHEAD < _BPW)
        def _():
            ns = (j + _LOOKAHEAD) % _NSLOT

            @pl.when(j + _LOOKAHEAD >= _NSLOT)
            def _():
                writes_wait(j + _LOOKAHEAD - _NSLOT, ns)

            gathers(j + _LOOKAHEAD, ns)

        return carry

    lax.fori_loop(0, _BPW, body, 0)

    for j in range(_BPW - _NSLOT, _BPW):
        writes_wait(j, j % _NSLOT)


@functools.partial(jax.jit, static_argnums=())
def _run(idx, embedding, position_embedding):
    mesh = plsc.VectorSubcoreMesh(core_axis_name="c", subcore_axis_name="s")
    kern = pl.kernel(
        _sc_body,
        mesh=mesh,
        compiler_params=pltpu.CompilerParams(use_tc_tiling_on_sc=False),
        out_type=jax.ShapeDtypeStruct((_BATCH * _SEQ, 2 * _HIDDEN), jnp.float32),
        scratch_types=[
            pltpu.VMEM((_BPW, 2, _HSEQ), jnp.int32),
            pltpu.VMEM((_SEQ, _HIDDEN), jnp.float32),
            pltpu.VMEM((_NSLOT, _SEQ, _HIDDEN), jnp.float32),
            pltpu.SemaphoreType.DMA((_NSLOT,)),
            pltpu.SemaphoreType.DMA((_NSLOT,)),
            pltpu.SemaphoreType.DMA((_NSLOT,)),
        ],
    )
    return kern(idx, embedding, position_embedding)


def kernel(inputs, embedding, position_embedding):
    batch, seq = inputs.shape
    hidden = embedding.shape[1]
    idx = inputs.reshape(_NW, _BPW, 2, _HSEQ)
    out = _run(idx, embedding, position_embedding)
    return out.reshape(batch, seq, 2 * hidden)


# speedup vs baseline: 4.5898x; 1.0089x over previous
import functools

import jax
import jax.numpy as jnp
from jax import lax
from jax.experimental import pallas as pl
from jax.experimental.pallas import tpu as pltpu
from jax.experimental.pallas import tpu_sc as plsc

_VOCAB = 1000000
_HIDDEN = 64
_BATCH = 4096
_SEQ = 200

_info = plsc.get_sparse_core_info()
_NC, _NS = _info.num_cores, _info.num_subcores
_NW = _NC * _NS
_BPW = _BATCH // _NW
_S0 = 104
_S1 = _SEQ - _S0
_NSLOT = 4
_LOOKAHEAD = 2


def _sc_body(idx_hbm, emb_hbm, pos_hbm, out_hbm,
             idx_v, pos_v, rows, gsem, wsem, psem):
    wid = lax.axis_index("s") * _NC + lax.axis_index("c")
    pltpu.sync_copy(idx_hbm.at[pl.ds(wid * _BPW, _BPW), :], idx_v)
    pltpu.sync_copy(pos_hbm.at[pl.ds(0, _SEQ)], pos_v)

    def gathers(j, slot):
        pltpu.make_async_copy(emb_hbm.at[idx_v.at[j, pl.ds(0, _S0)]],
                              rows.at[slot, pl.ds(0, _S0)],
                              gsem.at[slot]).start()
        pltpu.make_async_copy(emb_hbm.at[idx_v.at[j, pl.ds(_S0, _S1)]],
                              rows.at[slot, pl.ds(_S0, _S1)],
                              gsem.at[slot]).start()

    def wait_gathers(j, slot):
        pltpu.make_async_copy(emb_hbm.at[idx_v.at[j, pl.ds(0, _S0)]],
                              rows.at[slot, pl.ds(0, _S0)],
                              gsem.at[slot]).wait()
        pltpu.make_async_copy(emb_hbm.at[idx_v.at[j, pl.ds(_S0, _S1)]],
                              rows.at[slot, pl.ds(_S0, _S1)],
                              gsem.at[slot]).wait()

    def writes_start(j, slot):
        b = wid * _BPW + j
        pltpu.make_async_copy(
            pos_v, out_hbm.at[b, :, pl.ds(0, _HIDDEN)],
            psem.at[slot]).start()
        pltpu.make_async_copy(
            rows.at[slot],
            out_hbm.at[b, :, pl.ds(_HIDDEN, _HIDDEN)],
            wsem.at[slot]).start()

    def writes_wait(j, slot):
        b = wid * _BPW + j
        pltpu.make_async_copy(
            pos_v, out_hbm.at[b, :, pl.ds(0, _HIDDEN)],
            psem.at[slot]).wait()
        pltpu.make_async_copy(
            rows.at[slot],
            out_hbm.at[b, :, pl.ds(_HIDDEN, _HIDDEN)],
            wsem.at[slot]).wait()

    for j in range(_LOOKAHEAD):
        gathers(j, j % _NSLOT)

    def body(j, carry):
        slot = j % _NSLOT
        wait_gathers(j, slot)
        writes_start(j, slot)

        @pl.when(j + _LOOKAHEAD < _BPW)
        def _():
            ns = (j + _LOOKAHEAD) % _NSLOT

            @pl.when(j + _LOOKAHEAD >= _NSLOT)
            def _():
                writes_wait(j + _LOOKAHEAD - _NSLOT, ns)

            gathers(j + _LOOKAHEAD, ns)

        return carry

    lax.fori_loop(0, _BPW, body, 0)

    for j in range(_BPW - _NSLOT, _BPW):
        writes_wait(j, j % _NSLOT)


@functools.partial(jax.jit, static_argnums=())
def _run(idx, embedding, position_embedding):
    mesh = plsc.VectorSubcoreMesh(core_axis_name="c", subcore_axis_name="s")
    kern = pl.kernel(
        _sc_body,
        mesh=mesh,
        compiler_params=pltpu.CompilerParams(use_tc_tiling_on_sc=False),
        out_type=jax.ShapeDtypeStruct((_BATCH, _SEQ, 2 * _HIDDEN),
                                      jnp.float32),
        scratch_types=[
            pltpu.VMEM((_BPW, _SEQ), jnp.int32),
            pltpu.VMEM((_SEQ, _HIDDEN), jnp.float32),
            pltpu.VMEM((_NSLOT, _SEQ, _HIDDEN), jnp.float32),
            pltpu.SemaphoreType.DMA((_NSLOT,)),
            pltpu.SemaphoreType.DMA((_NSLOT,)),
            pltpu.SemaphoreType.DMA((_NSLOT,)),
        ],
    )
    return kern(idx, embedding, position_embedding)


def kernel(inputs, embedding, position_embedding):
    return _run(inputs, embedding, position_embedding)
